# Initial kernel scaffold; baseline (speedup 1.0000x reference)
import jax
import jax.numpy as jnp
from jax.experimental import pallas as pl


def kernel(pred_similarities, regions_mask, kernels_mask):
    raise NotImplementedError("write your pallas kernel here")


# speedup vs baseline: 1.9193x; 1.9193x over previous
import functools

import jax
import jax.numpy as jnp
from jax import lax
from jax.experimental import pallas as pl
from jax.experimental.pallas import tpu as pltpu
from jax.experimental.pallas import tpu_sc as plsc

_SIGMA = 3.0
_NUM_BLOCKS = 8
_BLK = 40
_OFF = 32


def _sc_block_sums(pred, km):
  mesh = plsc.VectorSubcoreMesh(core_axis_name="c", subcore_axis_name="s")

  @functools.partial(
      pl.kernel,
      mesh=mesh,
      out_type=jax.ShapeDtypeStruct((32, 8, 128), jnp.float32),
      scratch_types=[
          pltpu.VMEM((_BLK, 128), jnp.float32),
          pltpu.VMEM((_BLK, 128), jnp.int32),
          pltpu.VMEM((8, 128), jnp.float32),
          pltpu.SemaphoreType.DMA,
      ],
  )
  def body(pred_hbm, km_hbm, out_hbm, pbuf, kbuf, stage, sem):
    cid = lax.axis_index("c")
    sid = lax.axis_index("s")
    wid = cid * 16 + sid
    blk = wid // 4
    ch = lax.rem(wid, 4)
    r0 = 64 + (blk // 4) * 256
    ct = lax.rem(blk, 4) * 128

    cp_p = pltpu.async_copy(
        pred_hbm.at[ch, pl.ds(r0, _BLK), pl.ds(ct, 128)], pbuf, sem)
    cp_k = pltpu.async_copy(
        km_hbm.at[pl.ds(r0, _BLK), pl.ds(ct, 128)], kbuf, sem)
    cp_p.wait()
    cp_k.wait()

    lane = jnp.arange(16, dtype=jnp.int32)
    hi8 = lane >= 8
    zero = jnp.zeros((16,), jnp.float32)
    one = jnp.ones((16,), jnp.float32)
    acc = zero
    cnt = zero
    for r in range(_BLK):
      p0 = pbuf[r, pl.ds(_OFF, 16)]
      p1 = pbuf[r, pl.ds(_OFF + 16, 16)]
      p2 = pbuf[r, pl.ds(_OFF + 24, 16)]
      m0 = kbuf[r, pl.ds(_OFF, 16)] != 0
      m1 = kbuf[r, pl.ds(_OFF + 16, 16)] != 0
      m2 = (kbuf[r, pl.ds(_OFF + 24, 16)] != 0) & hi8
      acc = acc + jnp.where(m0, p0, zero) + jnp.where(m1, p1, zero)
      acc = acc + jnp.where(m2, p2, zero)
      cnt = cnt + jnp.where(m0, one, zero) + jnp.where(m1, one, zero)
      cnt = cnt + jnp.where(m2, one, zero)

    stage[0, pl.ds(0, 16)] = acc
    stage[0, pl.ds(16, 16)] = cnt
    pltpu.sync_copy(stage, out_hbm.at[wid])

  return body(pred, km)


def _tc_finish(partials):
  def body(p_ref, o_ref):
    x = p_ref[...][:, 0, :]
    s = x[:, 0:16]
    c = x[:, 16:32]
    S = jnp.sum(s, axis=1, keepdims=True)
    C = jnp.sum(c, axis=1, keepdims=True)
    q = jnp.sum((S * S).reshape(8, 4), axis=1, keepdims=True)
    n = C.reshape(8, 4)[:, 0:1]
    t = q * n
    m = t + t.reshape(1, 8)
    d = jnp.maximum(_SIGMA - jnp.sqrt(m), 0.0)
    term = jnp.log(d * d + 1.0)
    rows = lax.broadcasted_iota(jnp.int32, (8, 8), 0)
    cols = lax.broadcasted_iota(jnp.int32, (8, 8), 1)
    upper = rows < cols
    scale = (_NUM_BLOCKS - 1) / _NUM_BLOCKS
    o_ref[0, 0] = scale * jnp.sum(jnp.where(upper, term, 0.0))

  return pl.pallas_call(
      body,
      out_shape=jax.ShapeDtypeStruct((1, 1), jnp.float32),
      out_specs=pl.BlockSpec(memory_space=pltpu.SMEM),
  )(partials)


@jax.jit
def kernel(pred_similarities, regions_mask, kernels_mask):
  del regions_mask
  partials = _sc_block_sums(pred_similarities, kernels_mask)
  out = _tc_finish(partials)
  return out[0, 0]


# speedup vs baseline: 2.0532x; 1.0697x over previous
import functools

import jax
import jax.numpy as jnp
from jax import lax
from jax.experimental import pallas as pl
from jax.experimental.pallas import tpu as pltpu
from jax.experimental.pallas import tpu_sc as plsc

_SIGMA = 3.0
_BLK = 40
_OFF = 32
_LN2 = 0.6931471805599453
_SQRT2 = 1.4142135623730951


def _newton_sqrt(x):
  i = lax.bitcast_convert_type(x, jnp.int32)
  y = lax.bitcast_convert_type(0x5F3759DF - (i >> 1), jnp.float32)
  for _ in range(3):
    y = y * (1.5 - 0.5 * x * y * y)
  return x * y


def _log(x):
  bits = lax.bitcast_convert_type(x, jnp.int32)
  e = (bits >> 23) - 127
  m = lax.bitcast_convert_type((bits & 0x007FFFFF) | 0x3F800000, jnp.float32)
  big = m > _SQRT2
  m = jnp.where(big, 0.5 * m, m)
  ef = e.astype(jnp.float32) + jnp.where(big, 1.0, 0.0)
  u = (m - 1.0) / (m + 1.0)
  u2 = u * u
  p = 1.0 + u2 * (1.0 / 3.0 + u2 * (1.0 / 5.0 + u2 * (1.0 / 7.0 + u2 / 9.0)))
  return ef * _LN2 + 2.0 * u * p


def _sc_loss(pred, km):
  mesh = plsc.VectorSubcoreMesh(
      core_axis_name="c", subcore_axis_name="s", num_cores=1, num_subcores=16)

  @functools.partial(
      pl.kernel,
      mesh=mesh,
      out_type=[
          jax.ShapeDtypeStruct((16, 16), jnp.float32),
          jax.ShapeDtypeStruct((16,), jnp.float32),
      ],
      compiler_params=pltpu.CompilerParams(needs_layout_passes=False),
      scratch_types=[
          pltpu.VMEM((_BLK, 128), jnp.float32),
          pltpu.VMEM((_BLK, 128), jnp.float32),
          pltpu.VMEM((_BLK, 128), jnp.int32),
          pltpu.VMEM((16,), jnp.float32),
          pltpu.VMEM((16,), jnp.float32),
          pltpu.VMEM((16, 16), jnp.float32),
          pltpu.VMEM((16,), jnp.float32),
          pltpu.VMEM((16,), jnp.float32),
          pltpu.SemaphoreType.DMA,
      ],
  )
  def body(pred_hbm, km_hbm, stage_hbm, out_hbm, pbufa, pbufb, kbuf, gbuf,
           sbuf, allbuf, wbuf, tbuf, sem):
    sid = lax.axis_index("s")
    blk = sid // 2
    cha = lax.rem(sid, 2) * 2
    r0 = 64 + (blk // 4) * 256
    ct = lax.rem(blk, 4) * 128

    cp_a = pltpu.async_copy(
        pred_hbm.at[cha, pl.ds(r0, _BLK), pl.ds(ct, 128)], pbufa, sem)
    cp_b = pltpu.async_copy(
        pred_hbm.at[cha + 1, pl.ds(r0, _BLK), pl.ds(ct, 128)], pbufb, sem)
    cp_k = pltpu.async_copy(
        km_hbm.at[pl.ds(r0, _BLK), pl.ds(ct, 128)], kbuf, sem)
    cp_a.wait()
    cp_b.wait()
    cp_k.wait()

    lane = jnp.arange(16, dtype=jnp.int32)
    hi8 = lane >= 8
    zero = jnp.zeros((16,), jnp.float32)
    one = jnp.ones((16,), jnp.float32)

    def lane_total(v):
      for k in (8, 4, 2, 1):
        gbuf[...] = v
        v = v + plsc.load_gather(gbuf, [jnp.bitwise_xor(lane, k)])
      return v

    a0 = a1 = a2 = zero
    b0 = b1 = b2 = zero
    c0 = c1 = c2 = zero
    for r in range(_BLK):
      m0 = kbuf[r, pl.ds(_OFF, 16)] != 0
      m1 = kbuf[r, pl.ds(_OFF + 16, 16)] != 0
      m2 = (kbuf[r, pl.ds(_OFF + 24, 16)] != 0) & hi8
      a0 = a0 + jnp.where(m0, pbufa[r, pl.ds(_OFF, 16)], zero)
      a1 = a1 + jnp.where(m1, pbufa[r, pl.ds(_OFF + 16, 16)], zero)
      a2 = a2 + jnp.where(m2, pbufa[r, pl.ds(_OFF + 24, 16)], zero)
      b0 = b0 + jnp.where(m0, pbufb[r, pl.ds(_OFF, 16)], zero)
      b1 = b1 + jnp.where(m1, pbufb[r, pl.ds(_OFF + 16, 16)], zero)
      b2 = b2 + jnp.where(m2, pbufb[r, pl.ds(_OFF + 24, 16)], zero)
      c0 = c0 + jnp.where(m0, one, zero)
      c1 = c1 + jnp.where(m1, one, zero)
      c2 = c2 + jnp.where(m2, one, zero)

    sa = lane_total(a0 + a1 + a2)
    sb = lane_total(b0 + b1 + b2)
    cn = lane_total(c0 + c1 + c2)
    sbuf[...] = (jnp.where(lane == blk, sa * sa + sb * sb, zero)
                 + jnp.where(lane == blk + 8, 0.5 * cn, zero))
    pltpu.sync_copy(sbuf, stage_hbm.at[sid])
    plsc.subcore_barrier()

    @pl.when(sid == 0)
    def _():
      pltpu.async_copy(stage_hbm, allbuf, sem).wait()
      w = allbuf[0, :]
      for s in range(1, 16):
        w = w + allbuf[s, :]
      wbuf[...] = w
      nv = plsc.load_gather(wbuf, [jnp.bitwise_or(lane, 8)])
      tbuf[...] = w * nv

      l = lane
      ia1 = jnp.where(l < 7, 0, jnp.where(l < 13, 1, 2))
      ib1 = l + jnp.where(l < 7, 1, jnp.where(l < 13, -5, -10))
      ia2 = jnp.where(
          l < 2, 2,
          jnp.where(l < 6, 3,
                    jnp.where(l < 9, 4,
                              jnp.where(l < 11, 5, jnp.where(l < 12, 6, 0)))))
      ib2 = jnp.where(
          l < 2, l + 6,
          jnp.where(l < 6, l + 2,
                    jnp.where(l < 9, l - 1,
                              jnp.where(l < 11, l - 3,
                                        jnp.where(l < 12, l - 4, 0)))))
      m1v = plsc.load_gather(tbuf, [ia1]) + plsc.load_gather(tbuf, [ib1])
      m2v = plsc.load_gather(tbuf, [ia2]) + plsc.load_gather(tbuf, [ib2])
      d1 = jnp.maximum(_SIGMA - _newton_sqrt(m1v), 0.0)
      d2 = jnp.maximum(_SIGMA - _newton_sqrt(m2v), 0.0)
      t1 = _log(d1 * d1 + 1.0)
      t2 = jnp.where(lane < 12, _log(d2 * d2 + 1.0), zero)
      total = lane_total(t1 + t2)
      sbuf[...] = total * (7.0 / 8.0)
      pltpu.sync_copy(sbuf, out_hbm)

  return body(pred, km)


@jax.jit
def kernel(pred_similarities, regions_mask, kernels_mask):
  del regions_mask
  _, out = _sc_loss(pred_similarities, kernels_mask)
  return out[0]
